# baseline (speedup 1.0000x reference)
import functools

import jax
import jax.numpy as jnp
from jax import lax
from jax.experimental import pallas as pl
from jax.experimental.pallas import tpu as pltpu
from jax.experimental.pallas import tpu_sc as plsc

B, L, D = 4096, 200, 64
N = B * L
NC, NS = 2, 16
NW = NC * NS
PER_W = N // NW
CHUNK = 256
SUB = 128
NSUB = CHUNK // SUB
NCHUNK = PER_W // CHUNK
NPAIR = NCHUNK // 2

_mesh = plsc.VectorSubcoreMesh(core_axis_name="c", subcore_axis_name="s")


@functools.partial(
    pl.kernel,
    mesh=_mesh,
    out_type=jax.ShapeDtypeStruct((N, D), jnp.float32),
    compiler_params=pltpu.CompilerParams(use_tc_tiling_on_sc=False),
    scratch_types=[
        pltpu.VMEM((2, CHUNK), jnp.int32),
        pltpu.VMEM((2, CHUNK), jnp.int32),
        pltpu.VMEM((2, CHUNK, D), jnp.float32),
        pltpu.VMEM((2, CHUNK, D), jnp.float32),
        pltpu.SemaphoreType.DMA,
        pltpu.SemaphoreType.DMA,
        pltpu.SemaphoreType.DMA,
        pltpu.SemaphoreType.DMA,
        pltpu.SemaphoreType.DMA,
        pltpu.SemaphoreType.DMA,
    ],
)
def _embed_kernel(x_hbm, seq_hbm, tab_hbm, pos_hbm, out_hbm,
                  tok_idx, pos_idx, tok_rows, pos_rows,
                  sem_in0, sem_in1, sem_out0, sem_out1, sem_idx0, sem_idx1):
    sem_in = (sem_in0, sem_in1)
    sem_out = (sem_out0, sem_out1)
    sem_idx = (sem_idx0, sem_idx1)
    wid = lax.axis_index("s") * NC + lax.axis_index("c")
    base0 = wid * PER_W

    def issue_gathers(gdyn, b):
        base = base0 + gdyn * CHUNK
        for j in range(NSUB):
            s = pl.ds(j * SUB, SUB)
            pltpu.async_copy(tab_hbm.at[tok_idx.at[b, s]],
                             tok_rows.at[b, s], sem_in[b])
            pltpu.async_copy(pos_hbm.at[pos_idx.at[b, s]],
                             pos_rows.at[b, s], sem_in[b])

    def drain_gathers(b):
        pltpu.make_async_copy(tab_hbm.at[pl.ds(0, CHUNK)],
                              tok_rows.at[b], sem_in[b]).wait()
        pltpu.make_async_copy(tab_hbm.at[pl.ds(0, CHUNK)],
                              pos_rows.at[b], sem_in[b]).wait()

    def issue_idx(gdyn, b):
        base = base0 + gdyn * CHUNK
        pltpu.async_copy(x_hbm.at[pl.ds(base, CHUNK)], tok_idx.at[b], sem_idx[b])
        pltpu.async_copy(seq_hbm.at[pl.ds(base, CHUNK)], pos_idx.at[b], sem_idx[b])

    def drain_idx(b):
        pltpu.make_async_copy(x_hbm.at[pl.ds(0, CHUNK)],
                              tok_idx.at[b], sem_idx[b]).wait()
        pltpu.make_async_copy(x_hbm.at[pl.ds(0, CHUNK)],
                              pos_idx.at[b], sem_idx[b]).wait()

    def issue_out(gdyn, b):
        base = base0 + gdyn * CHUNK
        pltpu.async_copy(tok_rows.at[b], out_hbm.at[pl.ds(base, CHUNK)], sem_out[b])

    def drain_out(b):
        pltpu.make_async_copy(tok_rows.at[b], out_hbm.at[pl.ds(0, CHUNK)],
                              sem_out[b]).wait()

    def add_chunk(b):
        def add_rows(r, carry):
            for k in range(4):
                row = r * 4 + k
                for c in range(D // 16):
                    sl = pl.ds(c * 16, 16)
                    tok_rows[b, row, sl] = tok_rows[b, row, sl] + pos_rows[b, row, sl]
            return carry
        lax.fori_loop(0, CHUNK // 4, add_rows, 0)

    pltpu.sync_copy(x_hbm.at[pl.ds(base0, CHUNK)], tok_idx.at[0])
    pltpu.sync_copy(seq_hbm.at[pl.ds(base0, CHUNK)], pos_idx.at[0])
    issue_gathers(0, 0)
    issue_idx(1, 1)

    def pair_body(g2, carry):
        for b in range(2):
            g = 2 * g2 + b
            nb = 1 - b
            if b == 0:
                drain_idx(nb)

                @pl.when(g2 >= 1)
                def _():
                    drain_out(nb)
                issue_gathers(g + 1, nb)
            else:
                @pl.when(g2 < NPAIR - 1)
                def _():
                    drain_idx(nb)
                    drain_out(nb)
                    issue_gathers(g + 1, nb)
            drain_gathers(b)

            @pl.when(g2 < NPAIR - 1)
            def _():
                issue_idx(g + 2, b)
            add_chunk(b)
            issue_out(g, b)
        return carry

    lax.fori_loop(0, NPAIR, pair_body, 0)
    drain_out(0)
    drain_out(1)


def kernel(x, seq_idx, embed_table, pos_table):
    x_flat = x.reshape(-1).astype(jnp.int32)
    seq_flat = seq_idx.reshape(-1).astype(jnp.int32)
    out = _embed_kernel(x_flat, seq_flat, embed_table, pos_table)
    return out.reshape(B, L, D)


# speedup vs baseline: 1.0753x; 1.0753x over previous
import functools

import jax
import jax.numpy as jnp
from jax import lax
from jax.experimental import pallas as pl
from jax.experimental.pallas import tpu as pltpu
from jax.experimental.pallas import tpu_sc as plsc

B, L, D = 4096, 200, 64
MAX_POS = 252
N = B * L
NC, NS = 2, 16
NW = NC * NS
PER_W = N // NW
CHUNK = 512
SUB = 128
NSUB = CHUNK // SUB
NCHUNK = PER_W // CHUNK
NPAIR = NCHUNK // 2

_mesh = plsc.VectorSubcoreMesh(core_axis_name="c", subcore_axis_name="s")


@functools.partial(
    pl.kernel,
    mesh=_mesh,
    out_type=jax.ShapeDtypeStruct((N, D), jnp.float32),
    compiler_params=pltpu.CompilerParams(use_tc_tiling_on_sc=False),
    scratch_types=[
        pltpu.VMEM((2, CHUNK), jnp.int32),
        pltpu.VMEM((2, CHUNK), jnp.int32),
        pltpu.VMEM((2, CHUNK, D), jnp.float32),
        pltpu.VMEM((MAX_POS, D), jnp.float32),
        pltpu.SemaphoreType.DMA,
        pltpu.SemaphoreType.DMA,
        pltpu.SemaphoreType.DMA,
        pltpu.SemaphoreType.DMA,
        pltpu.SemaphoreType.DMA,
        pltpu.SemaphoreType.DMA,
    ],
)
def _embed_kernel(x_hbm, seq_hbm, tab_hbm, pos_hbm, out_hbm,
                  tok_idx, pos_idx, tok_rows, pos_local,
                  sem_in0, sem_in1, sem_out0, sem_out1, sem_idx0, sem_idx1):
    sem_in = (sem_in0, sem_in1)
    sem_out = (sem_out0, sem_out1)
    sem_idx = (sem_idx0, sem_idx1)
    wid = lax.axis_index("s") * NC + lax.axis_index("c")
    base0 = wid * PER_W

    def issue_gathers(gdyn, b):
        del gdyn
        for j in range(NSUB):
            s = pl.ds(j * SUB, SUB)
            pltpu.async_copy(tab_hbm.at[tok_idx.at[b, s]],
                             tok_rows.at[b, s], sem_in[b])

    def drain_gathers(b):
        pltpu.make_async_copy(tab_hbm.at[pl.ds(0, CHUNK)],
                              tok_rows.at[b], sem_in[b]).wait()

    def issue_idx(gdyn, b):
        base = base0 + gdyn * CHUNK
        pltpu.async_copy(x_hbm.at[pl.ds(base, CHUNK)], tok_idx.at[b], sem_idx[b])
        pltpu.async_copy(seq_hbm.at[pl.ds(base, CHUNK)], pos_idx.at[b], sem_idx[b])

    def drain_idx(b):
        pltpu.make_async_copy(x_hbm.at[pl.ds(0, CHUNK)],
                              tok_idx.at[b], sem_idx[b]).wait()
        pltpu.make_async_copy(x_hbm.at[pl.ds(0, CHUNK)],
                              pos_idx.at[b], sem_idx[b]).wait()

    def issue_out(gdyn, b):
        base = base0 + gdyn * CHUNK
        pltpu.async_copy(tok_rows.at[b], out_hbm.at[pl.ds(base, CHUNK)], sem_out[b])

    def drain_out(b):
        pltpu.make_async_copy(tok_rows.at[b], out_hbm.at[pl.ds(0, CHUNK)],
                              sem_out[b]).wait()

    def add_chunk(b):
        def add_rows(r, carry):
            pvec = pos_idx[b, pl.ds(r * 16, 16)]
            for k in range(16):
                row = r * 16 + k
                p = pvec[k]
                for c in range(D // 16):
                    sl = pl.ds(c * 16, 16)
                    tok_rows[b, row, sl] = tok_rows[b, row, sl] + pos_local[p, sl]
            return carry
        lax.fori_loop(0, CHUNK // 16, add_rows, 0)

    pltpu.sync_copy(pos_hbm, pos_local)
    pltpu.sync_copy(x_hbm.at[pl.ds(base0, CHUNK)], tok_idx.at[0])
    pltpu.sync_copy(seq_hbm.at[pl.ds(base0, CHUNK)], pos_idx.at[0])
    issue_gathers(0, 0)
    issue_idx(1, 1)

    def pair_body(g2, carry):
        for b in range(2):
            g = 2 * g2 + b
            nb = 1 - b
            if b == 0:
                drain_idx(nb)

                @pl.when(g2 >= 1)
                def _():
                    drain_out(nb)
                issue_gathers(g + 1, nb)
            else:
                @pl.when(g2 < NPAIR - 1)
                def _():
                    drain_idx(nb)
                    drain_out(nb)
                    issue_gathers(g + 1, nb)
            drain_gathers(b)

            @pl.when(g2 < NPAIR - 1)
            def _():
                issue_idx(g + 2, b)
            add_chunk(b)
            issue_out(g, b)
        return carry

    lax.fori_loop(0, NPAIR, pair_body, 0)
    drain_out(0)
    drain_out(1)


def kernel(x, seq_idx, embed_table, pos_table):
    x_flat = x.reshape(-1).astype(jnp.int32)
    seq_flat = seq_idx.reshape(-1).astype(jnp.int32)
    out = _embed_kernel(x_flat, seq_flat, embed_table, pos_table)
    return out.reshape(B, L, D)
